# baseline (speedup 1.0000x reference)
import functools

import jax
import jax.numpy as jnp
from jax import lax
from jax.experimental import pallas as pl
from jax.experimental.pallas import tpu as pltpu
from jax.experimental.pallas import tpu_sc as plsc

NC = 2
NS = 16
NW = NC * NS
CHUNK = 128
NBUF = 8


@functools.partial(jax.jit, static_argnames=("n_chunks", "d"))
def _sc_gather(idx, weight, *, n_chunks, d):
    per_w = n_chunks * CHUNK
    n_total = NW * per_w

    mesh = plsc.VectorSubcoreMesh(core_axis_name="c", subcore_axis_name="s")

    @functools.partial(
        pl.kernel,
        out_type=jax.ShapeDtypeStruct((n_total, d), jnp.float32),
        mesh=mesh,
        compiler_params=pltpu.CompilerParams(use_tc_tiling_on_sc=False),
        scratch_types=[
            pltpu.VMEM((n_chunks, CHUNK), jnp.int32),
            pltpu.VMEM((NBUF, CHUNK, d), jnp.float32),
            pltpu.SemaphoreType.DMA,
        ]
        + [pltpu.SemaphoreType.DMA] * NBUF
        + [pltpu.SemaphoreType.DMA] * NBUF,
    )
    def run(idx_hbm, tbl_hbm, out_hbm, idx_v, ring_v, sem_i, *sems):
        gsems = sems[:NBUF]
        ssems = sems[NBUF:]
        wid = lax.axis_index("s") * NC + lax.axis_index("c")
        base = wid * per_w

        copy_i = pltpu.make_async_copy(idx_hbm.at[wid], idx_v, sem_i)
        copy_i.start()
        copy_i.wait()

        def gather_start(i, b):
            pltpu.async_copy(tbl_hbm.at[idx_v.at[i]], ring_v.at[b], gsems[b])

        def gather_wait(b):
            pltpu.make_async_copy(
                tbl_hbm.at[idx_v.at[0]], ring_v.at[b], gsems[b]
            ).wait()

        def scatter_start(i, b):
            pltpu.async_copy(
                ring_v.at[b],
                out_hbm.at[pl.ds(base + i * CHUNK, CHUNK)],
                ssems[b],
            )

        def scatter_wait(b):
            pltpu.make_async_copy(
                ring_v.at[b], out_hbm.at[pl.ds(base, CHUNK)], ssems[b]
            ).wait()

        n_groups = n_chunks // NBUF

        for b in range(NBUF):
            gather_start(b, b)

        def group_body(g, carry):
            for b in range(NBUF):
                gather_wait(b)
                scatter_start(g * NBUF + b, b)
            for b in range(NBUF):
                scatter_wait(b)
                gather_start((g + 1) * NBUF + b, b)
            return carry

        lax.fori_loop(0, n_groups - 1, group_body, 0, unroll=False)

        last = (n_groups - 1) * NBUF
        for b in range(NBUF):
            gather_wait(b)
            scatter_start(last + b, b)
        for b in range(NBUF):
            scatter_wait(b)

    return run(idx, weight)


def kernel(x, weight):
    b_sz, l_sz = x.shape
    v, d = weight.shape
    n = b_sz * l_sz
    per_w = n // NW
    n_chunks = per_w // CHUNK
    idx = jnp.clip(x.astype(jnp.int32), 0, v - 1).reshape(NW, n_chunks, CHUNK)
    out = _sc_gather(idx, weight, n_chunks=n_chunks, d=d)
    return out.reshape(b_sz, l_sz, d)
